# baseline (speedup 1.0000x reference)
import functools

import jax
import jax.numpy as jnp
from jax.experimental import pallas as pl
from jax.experimental.pallas import tpu as pltpu

_ITERS = 14
_SPLIT = 14


def _split_reduce(arr, op, combine, h):
    step = h // _SPLIT
    parts = [
        op(arr[:, j * step:(j + 1) * step, :], axis=(1, 2), keepdims=True)
        for j in range(_SPLIT)
    ]
    while len(parts) > 1:
        nxt = [combine(parts[i], parts[i + 1])
               for i in range(0, len(parts) - 1, 2)]
        if len(parts) % 2:
            nxt.append(parts[-1])
        parts = nxt
    return parts[0]


def _pool_body(t_ref, x_ref, o_ref, *, k, h, iters):
    xb = x_ref[...]
    maxv = _split_reduce(xb, jnp.max, jnp.maximum, h)
    minv = _split_reduce(xb, jnp.min, jnp.minimum, h)
    kf = jnp.float32(k)

    def step(_, carry):
        lo, hi = carry
        mid = 0.5 * (lo + hi)
        cnt = _split_reduce(jnp.where(xb >= mid, 1.0, 0.0), jnp.sum, jnp.add, h)
        ok = cnt >= kf
        return jnp.where(ok, mid, lo), jnp.where(ok, hi, mid)

    lo, _ = jax.lax.fori_loop(0, iters, step, (minv, maxv))
    t = lo
    ge = xb >= t
    cnt_ge = _split_reduce(jnp.where(ge, 1.0, 0.0), jnp.sum, jnp.add, h)
    sum_ge = _split_reduce(jnp.where(ge, xb, 0.0), jnp.sum, jnp.add, h)
    topk_sum = sum_ge - (cnt_ge - kf) * t
    avg = topk_sum / kf

    denom = maxv + 1e-6
    s = jnp.minimum(t / denom, maxv / denom)
    ts = jax.nn.sigmoid(t_ref[0, 0])
    logits = (s - ts) / 0.1
    gate_soft = jax.nn.sigmoid(logits)
    gate_hard = (logits >= 0).astype(jnp.float32)
    gate = (gate_hard - gate_soft) + gate_soft
    pooled = gate * maxv + (1.0 - gate) * avg
    o_ref[...] = pooled[:, :, 0]


def kernel(x, T):
    B, C, H, W = x.shape
    n = H * W
    k = max(1, int(n * 0.1))
    rows = B * C
    r_blk = 32
    assert rows % r_blk == 0 and H % _SPLIT == 0
    xr = x.reshape(rows, H, W)
    t2 = jnp.reshape(T, (1, 1)).astype(jnp.float32)

    out = pl.pallas_call(
        functools.partial(_pool_body, k=k, h=H, iters=_ITERS),
        grid=(rows // r_blk,),
        in_specs=[
            pl.BlockSpec(memory_space=pltpu.SMEM),
            pl.BlockSpec((r_blk, H, W), lambda i: (i, 0, 0)),
        ],
        out_specs=pl.BlockSpec((r_blk, 1), lambda i: (i, 0)),
        out_shape=jax.ShapeDtypeStruct((rows, 1), jnp.float32),
        compiler_params=pltpu.CompilerParams(
            dimension_semantics=("arbitrary",),
        ),
    )(t2, xr)
    return out.reshape(B, C)
